# baseline (speedup 1.0000x reference)
import functools

import jax
import jax.numpy as jnp
from jax import lax
from jax.experimental import pallas as pl
from jax.experimental.pallas import tpu as pltpu
from jax.experimental.pallas import tpu_sc as plsc

B, T, F = 16384, 200, 14
STEPS = 20
SROWS = 24
NJ = 4
NC, NS, L = 2, 16, 16
NW = NC * NS
BW = B // NW
BCH = 128
NCH = BW // BCH
NGR = BCH // L


def _sc_phase1(x_t, w16):
    mesh = plsc.VectorSubcoreMesh(
        core_axis_name="c", subcore_axis_name="s",
        num_cores=NC, num_subcores=NS)

    @functools.partial(
        pl.kernel,
        out_type=jax.ShapeDtypeStruct((NJ, STEPS, B), jnp.float32),
        mesh=mesh,
        scratch_types=[
            pltpu.VMEM((NJ, SROWS, BCH), jnp.float32),
            pltpu.VMEM((NJ, SROWS, BCH), jnp.float32),
            pltpu.VMEM((9, 8, BCH), jnp.float32),
            pltpu.VMEM((9, 8, BCH), jnp.float32),
            pltpu.VMEM((NJ, SROWS, BCH), jnp.float32),
            pltpu.VMEM((NJ, SROWS, BCH), jnp.float32),
            pltpu.VMEM((L,), jnp.float32),
            pltpu.SemaphoreType.DMA,
            pltpu.SemaphoreType.DMA,
            pltpu.SemaphoreType.DMA,
            pltpu.SemaphoreType.DMA,
            pltpu.SemaphoreType.DMA,
            pltpu.SemaphoreType.DMA,
        ],
        compiler_params=pltpu.CompilerParams(
            use_tc_tiling_on_sc=True, needs_layout_passes=False),
    )
    def k(x_hbm, w_hbm, out_hbm, e0, e1, x0, x1, a0, a1, wv,
          se0, se1, sx0, sx1, so0, so1):
        wid = lax.axis_index("s") * NC + lax.axis_index("c")
        base0 = wid * BW

        def in_copies(base, ev, xv, sem_e, sem_x):
            ce = pltpu.async_copy(
                x_hbm.at[pl.ds(10, NJ), pl.ds(0, SROWS), pl.ds(base, BCH)],
                ev, sem_e)
            cx = pltpu.async_copy(
                x_hbm.at[pl.ds(0, 9), pl.ds(T - 8, 8), pl.ds(base, BCH)],
                xv, sem_x)
            return ce, cx

        pltpu.sync_copy(w_hbm, wv)
        wvec = wv[...]
        lanes = lax.iota(jnp.int32, L)

        def splat_i(v):
            return jnp.full((L,), v, jnp.int32)

        def compute(early_v, xl_v, ahat_v):
            @plsc.parallel_loop(0, NGR)
            def group_body(g):
                sl = pl.ds(g * L, L)
                col = lanes + g * L

                def xl_feat(f):
                    return xl_v[f, 7, sl]

                d = xl_feat(0)
                dsums = []
                for jj in range(NJ):
                    if jj > 0:
                        d = d + xl_feat(jj)
                    dsums.append(d)
                for jj in range(NJ):
                    denom = wvec + xl_feat(5 + jj) * jnp.float32(25.0)
                    td = dsums[jj] * jnp.float32(150.0) / denom
                    fshift = td * jnp.float32(10.0)
                    tr = fshift.astype(jnp.int32)
                    kk = jnp.where(fshift > tr.astype(jnp.float32),
                                   tr + 1, tr)
                    for i in range(STEPS):
                        idx = jnp.maximum(splat_i(i) - kk, 0)
                        val = plsc.load_gather(
                            early_v, [splat_i(jj), idx, col])
                        ahat_v[jj, i, sl] = val

        ebufs, xbufs, abufs = (e0, e1), (x0, x1), (a0, a1)
        esems, xsems, osems = (se0, se1), (sx0, sx1), (so0, so1)
        incps = {}
        outcps = {}
        for c in range(2):
            incps[c] = in_copies(base0 + c * BCH, ebufs[c], xbufs[c],
                                 esems[c], xsems[c])
        for c in range(NCH):
            p = c % 2
            ce, cx = incps[c]
            ce.wait()
            cx.wait()
            if c >= 2:
                outcps[c - 2].wait()
            compute(ebufs[p], xbufs[p], abufs[p])
            outcps[c] = pltpu.async_copy(
                abufs[p].at[:, pl.ds(0, STEPS), :],
                out_hbm.at[:, :, pl.ds(base0 + c * BCH, BCH)],
                osems[p])
            if c + 2 < NCH:
                incps[c + 2] = in_copies(base0 + (c + 2) * BCH, ebufs[p],
                                         xbufs[p], esems[p], xsems[p])
        outcps[NCH - 2].wait()
        outcps[NCH - 1].wait()

    return k(x_t, w16)


def _tc_phase2(ahat):
    BB = 4096
    NB = B // BB

    def body(a0, a1, a2, a3, out_ref, facc):
        refs = (a0, a1, a2, a3)
        p = pl.program_id(0)
        bb = pl.program_id(1)

        @pl.when(p == 0)
        def _():
            for jj in range(NJ):
                a = refs[jj][0]
                m = jnp.max((a != 0.0).astype(jnp.float32),
                            axis=1, keepdims=True)
                prev = jnp.where(bb == 0, jnp.zeros((STEPS, 1), jnp.float32),
                                 facc[0:STEPS, jj:jj + 1])
                facc[0:STEPS, jj:jj + 1] = jnp.maximum(prev, m)
            out_ref[...] = jnp.zeros_like(out_ref)

        @pl.when(p == 1)
        def _():
            res = jnp.zeros((STEPS, BB), jnp.float32)
            for jj in reversed(range(NJ)):
                fl = facc[0:STEPS, jj:jj + 1] > 0.0
                res = jnp.where(fl, refs[jj][0], res)
            out_ref[...] = res

    return pl.pallas_call(
        body,
        grid=(2, NB),
        in_specs=[
            pl.BlockSpec((1, STEPS, BB), lambda p, b, jj=jj: (jj, 0, b))
            for jj in range(NJ)
        ],
        out_specs=pl.BlockSpec((STEPS, BB), lambda p, b: (0, b)),
        out_shape=jax.ShapeDtypeStruct((STEPS, B), jnp.float32),
        scratch_shapes=[pltpu.VMEM((SROWS, 128), jnp.float32)],
    )(ahat, ahat, ahat, ahat)


def kernel(vi, delta_y, v_previous, x_input, w):
    x_t = jnp.transpose(x_input, (2, 1, 0))
    w16 = jnp.full((L,), w, jnp.float32)
    ahat = _sc_phase1(x_t, w16)
    out_t = _tc_phase2(ahat)
    return out_t.T


# speedup vs baseline: 1.1319x; 1.1319x over previous
import functools

import jax
import jax.numpy as jnp
from jax import lax
from jax.experimental import pallas as pl
from jax.experimental.pallas import tpu as pltpu
from jax.experimental.pallas import tpu_sc as plsc

B, T, F = 16384, 200, 14
STEPS = 20
SROWS = 24
NJ = 4
NC, NS, L = 2, 16, 16
NW = NC * NS
BW = B // NW
BCH = 128
NCH = BW // BCH
NGR = BCH // L


def _sc_phase1(x_t, w16):
    mesh = plsc.VectorSubcoreMesh(
        core_axis_name="c", subcore_axis_name="s",
        num_cores=NC, num_subcores=NS)

    @functools.partial(
        pl.kernel,
        out_type=jax.ShapeDtypeStruct((NJ, STEPS, B), jnp.float32),
        mesh=mesh,
        scratch_types=[
            pltpu.VMEM((NJ, SROWS, BCH), jnp.float32),
            pltpu.VMEM((NJ, SROWS, BCH), jnp.float32),
            pltpu.VMEM((9, 8, BCH), jnp.float32),
            pltpu.VMEM((9, 8, BCH), jnp.float32),
            pltpu.VMEM((NJ, SROWS, BCH), jnp.float32),
            pltpu.VMEM((NJ, SROWS, BCH), jnp.float32),
            pltpu.VMEM((L,), jnp.float32),
            pltpu.SemaphoreType.DMA,
            pltpu.SemaphoreType.DMA,
            pltpu.SemaphoreType.DMA,
            pltpu.SemaphoreType.DMA,
            pltpu.SemaphoreType.DMA,
            pltpu.SemaphoreType.DMA,
        ],
        compiler_params=pltpu.CompilerParams(
            use_tc_tiling_on_sc=True, needs_layout_passes=False),
    )
    def k(x_hbm, w_hbm, out_hbm, e0, e1, x0, x1, a0, a1, wv,
          se0, se1, sx0, sx1, so0, so1):
        wid = lax.axis_index("s") * NC + lax.axis_index("c")
        base0 = wid * BW

        def in_copies(base, ev, xv, sem_e, sem_x):
            ce = pltpu.async_copy(
                x_hbm.at[pl.ds(10, NJ), pl.ds(0, SROWS), pl.ds(base, BCH)],
                ev, sem_e)
            cx = pltpu.async_copy(
                x_hbm.at[pl.ds(0, 9), pl.ds(T - 8, 8), pl.ds(base, BCH)],
                xv, sem_x)
            return ce, cx

        pltpu.sync_copy(w_hbm, wv)
        wvec = wv[...]
        lanes = lax.iota(jnp.int32, L)

        def splat_i(v):
            return jnp.full((L,), v, jnp.int32)

        def compute(early_v, xl_v, ahat_v):
            def group_body(g, carry):
                sl = pl.ds(g * L, L)
                col = lanes + g * L

                def xl_feat(f):
                    return xl_v[f, 7, sl]

                d = xl_feat(0)
                dsums = []
                for jj in range(NJ):
                    if jj > 0:
                        d = d + xl_feat(jj)
                    dsums.append(d)
                kks = []
                for jj in range(NJ):
                    denom = wvec + xl_feat(5 + jj) * jnp.float32(25.0)
                    td = dsums[jj] * jnp.float32(150.0) / denom
                    fshift = td * jnp.float32(10.0)
                    tr = fshift.astype(jnp.int32)
                    kks.append(jnp.where(fshift > tr.astype(jnp.float32),
                                         tr + 1, tr))

                def i_body(i, carry2):
                    iv = jnp.full((L,), i, jnp.int32)
                    for jj in range(NJ):
                        idx = jnp.maximum(iv - kks[jj], 0)
                        val = plsc.load_gather(
                            early_v, [splat_i(jj), idx, col])
                        plsc.store_scatter(
                            ahat_v, [splat_i(jj), iv, col], val)
                    return carry2

                lax.fori_loop(0, STEPS, i_body, 0)
                return carry

            lax.fori_loop(0, NGR, group_body, 0)

        ebufs, xbufs, abufs = (e0, e1), (x0, x1), (a0, a1)
        esems, xsems, osems = (se0, se1), (sx0, sx1), (so0, so1)
        incps = {}
        outcps = {}
        for c in range(2):
            incps[c] = in_copies(base0 + c * BCH, ebufs[c], xbufs[c],
                                 esems[c], xsems[c])
        for c in range(NCH):
            p = c % 2
            ce, cx = incps[c]
            ce.wait()
            cx.wait()
            if c >= 2:
                outcps[c - 2].wait()
            compute(ebufs[p], xbufs[p], abufs[p])
            outcps[c] = pltpu.async_copy(
                abufs[p].at[:, pl.ds(0, STEPS), :],
                out_hbm.at[:, :, pl.ds(base0 + c * BCH, BCH)],
                osems[p])
            if c + 2 < NCH:
                incps[c + 2] = in_copies(base0 + (c + 2) * BCH, ebufs[p],
                                         xbufs[p], esems[p], xsems[p])
        outcps[NCH - 2].wait()
        outcps[NCH - 1].wait()

    return k(x_t, w16)


def _tc_phase2(ahat):
    BB = 4096
    NB = B // BB

    def body(a0, a1, a2, a3, out_ref, facc):
        refs = (a0, a1, a2, a3)
        p = pl.program_id(0)
        bb = pl.program_id(1)

        @pl.when(p == 0)
        def _():
            for jj in range(NJ):
                a = refs[jj][0]
                m = jnp.max((a != 0.0).astype(jnp.float32),
                            axis=1, keepdims=True)
                prev = jnp.where(bb == 0, jnp.zeros((STEPS, 1), jnp.float32),
                                 facc[0:STEPS, jj:jj + 1])
                facc[0:STEPS, jj:jj + 1] = jnp.maximum(prev, m)
            out_ref[...] = jnp.zeros_like(out_ref)

        @pl.when(p == 1)
        def _():
            res = jnp.zeros((STEPS, BB), jnp.float32)
            for jj in reversed(range(NJ)):
                fl = facc[0:STEPS, jj:jj + 1] > 0.0
                res = jnp.where(fl, refs[jj][0], res)
            out_ref[...] = res

    return pl.pallas_call(
        body,
        grid=(2, NB),
        in_specs=[
            pl.BlockSpec((1, STEPS, BB), lambda p, b, jj=jj: (jj, 0, b))
            for jj in range(NJ)
        ],
        out_specs=pl.BlockSpec((STEPS, BB), lambda p, b: (0, b)),
        out_shape=jax.ShapeDtypeStruct((STEPS, B), jnp.float32),
        scratch_shapes=[pltpu.VMEM((SROWS, 128), jnp.float32)],
    )(ahat, ahat, ahat, ahat)


def kernel(vi, delta_y, v_previous, x_input, w):
    x_t = jnp.transpose(x_input, (2, 1, 0))
    w16 = jnp.full((L,), w, jnp.float32)
    ahat = _sc_phase1(x_t, w16)
    out_t = _tc_phase2(ahat)
    return out_t.T


# speedup vs baseline: 1.1912x; 1.0524x over previous
import functools

import jax
import jax.numpy as jnp
from jax import lax
from jax.experimental import pallas as pl
from jax.experimental.pallas import tpu as pltpu
from jax.experimental.pallas import tpu_sc as plsc

B, T, F = 16384, 200, 14
STEPS = 20
SROWS = 24
NJ = 4
NC, NS, L = 2, 16, 16
NW = NC * NS
BW = B // NW
BCH = 128
NCH = BW // BCH
NGR = BCH // L


def _sc_phase1(x_t, w16):
    mesh = plsc.VectorSubcoreMesh(
        core_axis_name="c", subcore_axis_name="s",
        num_cores=NC, num_subcores=NS)

    @functools.partial(
        pl.kernel,
        out_type=jax.ShapeDtypeStruct((NJ, STEPS, B), jnp.float32),
        mesh=mesh,
        scratch_types=[
            pltpu.VMEM((NJ, SROWS, BCH), jnp.float32),
            pltpu.VMEM((NJ, SROWS, BCH), jnp.float32),
            pltpu.VMEM((9, 8, BCH), jnp.float32),
            pltpu.VMEM((9, 8, BCH), jnp.float32),
            pltpu.VMEM((NJ, SROWS, BCH), jnp.float32),
            pltpu.VMEM((NJ, SROWS, BCH), jnp.float32),
            pltpu.VMEM((L,), jnp.float32),
            pltpu.SemaphoreType.DMA,
            pltpu.SemaphoreType.DMA,
            pltpu.SemaphoreType.DMA,
            pltpu.SemaphoreType.DMA,
            pltpu.SemaphoreType.DMA,
            pltpu.SemaphoreType.DMA,
        ],
        compiler_params=pltpu.CompilerParams(
            use_tc_tiling_on_sc=True, needs_layout_passes=False),
    )
    def k(x_hbm, w_hbm, out_hbm, e0, e1, x0, x1, a0, a1, wv,
          se0, se1, sx0, sx1, so0, so1):
        wid = lax.axis_index("s") * NC + lax.axis_index("c")
        base0 = wid * BW

        def in_copies(base, ev, xv, sem_e, sem_x):
            ce = pltpu.async_copy(
                x_hbm.at[pl.ds(10, NJ), pl.ds(0, SROWS), pl.ds(base, BCH)],
                ev, sem_e)
            cx = pltpu.async_copy(
                x_hbm.at[pl.ds(0, 9), pl.ds(T - 8, 8), pl.ds(base, BCH)],
                xv, sem_x)
            return ce, cx

        pltpu.sync_copy(w_hbm, wv)
        wvec = wv[...]
        lanes = lax.iota(jnp.int32, L)

        def splat_i(v):
            return jnp.full((L,), v, jnp.int32)

        def compute(early_v, xl_v, ahat_v):
            def group_body(g, carry):
                sl = pl.ds(g * L, L)
                col = lanes + g * L

                def xl_feat(f):
                    return xl_v[f, 7, sl]

                d = xl_feat(0)
                dsums = []
                for jj in range(NJ):
                    if jj > 0:
                        d = d + xl_feat(jj)
                    dsums.append(d)
                kks = []
                for jj in range(NJ):
                    denom = wvec + xl_feat(5 + jj) * jnp.float32(25.0)
                    td = dsums[jj] * jnp.float32(150.0) / denom
                    fshift = td * jnp.float32(10.0)
                    tr = fshift.astype(jnp.int32)
                    kks.append(jnp.where(fshift > tr.astype(jnp.float32),
                                         tr + 1, tr))

                def i_body(i, carry2):
                    iv = jnp.full((L,), i, jnp.int32)
                    for jj in range(NJ):
                        idx = jnp.maximum(iv - kks[jj], 0)
                        val = plsc.load_gather(
                            early_v, [splat_i(jj), idx, col])
                        ahat_v[jj, i, sl] = val
                    return carry2

                lax.fori_loop(0, STEPS, i_body, 0)
                return carry

            lax.fori_loop(0, NGR, group_body, 0)

        ebufs, xbufs, abufs = (e0, e1), (x0, x1), (a0, a1)
        esems, xsems, osems = (se0, se1), (sx0, sx1), (so0, so1)
        incps = {}
        outcps = {}
        for c in range(2):
            incps[c] = in_copies(base0 + c * BCH, ebufs[c], xbufs[c],
                                 esems[c], xsems[c])
        for c in range(NCH):
            p = c % 2
            ce, cx = incps[c]
            ce.wait()
            cx.wait()
            if c >= 2:
                outcps[c - 2].wait()
            compute(ebufs[p], xbufs[p], abufs[p])
            outcps[c] = pltpu.async_copy(
                abufs[p].at[:, pl.ds(0, STEPS), :],
                out_hbm.at[:, :, pl.ds(base0 + c * BCH, BCH)],
                osems[p])
            if c + 2 < NCH:
                incps[c + 2] = in_copies(base0 + (c + 2) * BCH, ebufs[p],
                                         xbufs[p], esems[p], xsems[p])
        outcps[NCH - 2].wait()
        outcps[NCH - 1].wait()

    return k(x_t, w16)


def _tc_phase2(ahat):
    BB = 8192
    NB = B // BB

    def body(a0, a1, a2, a3, out_ref, facc):
        refs = (a0, a1, a2, a3)
        p = pl.program_id(0)
        bb = pl.program_id(1)

        @pl.when(p == 0)
        def _():
            for jj in range(NJ):
                a = refs[jj][0]
                m = jnp.max((a != 0.0).astype(jnp.float32),
                            axis=1, keepdims=True)
                prev = jnp.where(bb == 0, jnp.zeros((STEPS, 1), jnp.float32),
                                 facc[0:STEPS, jj:jj + 1])
                facc[0:STEPS, jj:jj + 1] = jnp.maximum(prev, m)
            out_ref[...] = jnp.zeros_like(out_ref)

        @pl.when(p == 1)
        def _():
            res = jnp.zeros((STEPS, BB), jnp.float32)
            for jj in reversed(range(NJ)):
                fl = facc[0:STEPS, jj:jj + 1] > 0.0
                res = jnp.where(fl, refs[jj][0], res)
            out_ref[...] = res

    return pl.pallas_call(
        body,
        grid=(2, NB),
        in_specs=[
            pl.BlockSpec((1, STEPS, BB), lambda p, b, jj=jj: (jj, 0, b))
            for jj in range(NJ)
        ],
        out_specs=pl.BlockSpec((STEPS, BB), lambda p, b: (0, b)),
        out_shape=jax.ShapeDtypeStruct((STEPS, B), jnp.float32),
        scratch_shapes=[pltpu.VMEM((SROWS, 128), jnp.float32)],
    )(ahat, ahat, ahat, ahat)


def kernel(vi, delta_y, v_previous, x_input, w):
    x_t = jnp.transpose(x_input, (2, 1, 0))
    w16 = jnp.full((L,), w, jnp.float32)
    ahat = _sc_phase1(x_t, w16)
    out_t = _tc_phase2(ahat)
    return out_t.T


# speedup vs baseline: 1.2040x; 1.0107x over previous
import functools

import jax
import jax.numpy as jnp
from jax import lax
from jax.experimental import pallas as pl
from jax.experimental.pallas import tpu as pltpu
from jax.experimental.pallas import tpu_sc as plsc

B, T, F = 16384, 200, 14
STEPS = 20
SROWS = 24
NJ = 4
NC, NS, L = 2, 16, 16
NW = NC * NS
BW = B // NW
BCH = 128
NCH = BW // BCH
NGR = BCH // L


def _sc_phase1(x_t, w16):
    mesh = plsc.VectorSubcoreMesh(
        core_axis_name="c", subcore_axis_name="s",
        num_cores=NC, num_subcores=NS)

    @functools.partial(
        pl.kernel,
        out_type=jax.ShapeDtypeStruct((NJ, STEPS, B), jnp.float32),
        mesh=mesh,
        scratch_types=[
            pltpu.VMEM((NJ, SROWS, BCH), jnp.float32),
            pltpu.VMEM((NJ, SROWS, BCH), jnp.float32),
            pltpu.VMEM((9, 8, BCH), jnp.float32),
            pltpu.VMEM((9, 8, BCH), jnp.float32),
            pltpu.VMEM((NJ, SROWS, BCH), jnp.float32),
            pltpu.VMEM((NJ, SROWS, BCH), jnp.float32),
            pltpu.VMEM((L,), jnp.float32),
            pltpu.SemaphoreType.DMA,
            pltpu.SemaphoreType.DMA,
            pltpu.SemaphoreType.DMA,
            pltpu.SemaphoreType.DMA,
            pltpu.SemaphoreType.DMA,
            pltpu.SemaphoreType.DMA,
        ],
        compiler_params=pltpu.CompilerParams(
            use_tc_tiling_on_sc=True, needs_layout_passes=False),
    )
    def k(x_hbm, w_hbm, out_hbm, e0, e1, x0, x1, a0, a1, wv,
          se0, se1, sx0, sx1, so0, so1):
        wid = lax.axis_index("s") * NC + lax.axis_index("c")
        base0 = wid * BW

        def in_copies(base, ev, xv, sem_e, sem_x):
            ce = pltpu.async_copy(
                x_hbm.at[pl.ds(10, NJ), pl.ds(0, SROWS), pl.ds(base, BCH)],
                ev, sem_e)
            cx = pltpu.async_copy(
                x_hbm.at[pl.ds(0, 9), pl.ds(T - 8, 8), pl.ds(base, BCH)],
                xv, sem_x)
            return ce, cx

        pltpu.sync_copy(w_hbm, wv)
        wvec = wv[...]
        lanes = lax.iota(jnp.int32, L)

        def splat_i(v):
            return jnp.full((L,), v, jnp.int32)

        def compute(early_v, xl_v, ahat_v):
            def group_body(g, carry):
                sl = pl.ds(g * L, L)
                col = lanes + g * L

                def xl_feat(f):
                    return xl_v[f, 7, sl]

                d = xl_feat(0)
                dsums = []
                for jj in range(NJ):
                    if jj > 0:
                        d = d + xl_feat(jj)
                    dsums.append(d)
                floors, bases = [], []
                for jj in range(NJ):
                    denom = wvec + xl_feat(5 + jj) * jnp.float32(25.0)
                    td = dsums[jj] * jnp.float32(150.0) / denom
                    fshift = td * jnp.float32(10.0)
                    tr = fshift.astype(jnp.int32)
                    kk = jnp.where(fshift > tr.astype(jnp.float32),
                                   tr + 1, tr)
                    fl = col + jj * (SROWS * BCH)
                    floors.append(fl)
                    bases.append(fl - lax.shift_left(kk, 7))

                zero = jnp.zeros((L,), jnp.int32)

                def i_body(i, carry2):
                    ib = i * BCH
                    for jj in range(NJ):
                        addr = jnp.maximum(floors[jj], bases[jj] + ib)
                        val = plsc.load_gather(
                            early_v, [zero, zero, addr])
                        ahat_v[jj, i, sl] = val
                    return carry2

                lax.fori_loop(0, STEPS, i_body, 0)
                return carry

            lax.fori_loop(0, NGR, group_body, 0)

        ebufs, xbufs, abufs = (e0, e1), (x0, x1), (a0, a1)
        esems, xsems, osems = (se0, se1), (sx0, sx1), (so0, so1)
        incps = {}
        outcps = {}
        for c in range(2):
            incps[c] = in_copies(base0 + c * BCH, ebufs[c], xbufs[c],
                                 esems[c], xsems[c])
        for c in range(NCH):
            p = c % 2
            ce, cx = incps[c]
            ce.wait()
            cx.wait()
            if c >= 2:
                outcps[c - 2].wait()
            compute(ebufs[p], xbufs[p], abufs[p])
            outcps[c] = pltpu.async_copy(
                abufs[p].at[:, pl.ds(0, STEPS), :],
                out_hbm.at[:, :, pl.ds(base0 + c * BCH, BCH)],
                osems[p])
            if c + 2 < NCH:
                incps[c + 2] = in_copies(base0 + (c + 2) * BCH, ebufs[p],
                                         xbufs[p], esems[p], xsems[p])
        outcps[NCH - 2].wait()
        outcps[NCH - 1].wait()

    return k(x_t, w16)


def _tc_phase2(ahat):
    BB = 8192
    NB = B // BB

    def body(a0, a1, a2, a3, out_ref, facc):
        refs = (a0, a1, a2, a3)
        p = pl.program_id(0)
        bb = pl.program_id(1)

        @pl.when(p == 0)
        def _():
            for jj in range(NJ):
                a = refs[jj][0]
                m = jnp.max((a != 0.0).astype(jnp.float32),
                            axis=1, keepdims=True)
                prev = jnp.where(bb == 0, jnp.zeros((STEPS, 1), jnp.float32),
                                 facc[0:STEPS, jj:jj + 1])
                facc[0:STEPS, jj:jj + 1] = jnp.maximum(prev, m)
            out_ref[...] = jnp.zeros_like(out_ref)

        @pl.when(p == 1)
        def _():
            res = jnp.zeros((STEPS, BB), jnp.float32)
            for jj in reversed(range(NJ)):
                fl = facc[0:STEPS, jj:jj + 1] > 0.0
                res = jnp.where(fl, refs[jj][0], res)
            out_ref[...] = res

    return pl.pallas_call(
        body,
        grid=(2, NB),
        in_specs=[
            pl.BlockSpec((1, STEPS, BB), lambda p, b, jj=jj: (jj, 0, b))
            for jj in range(NJ)
        ],
        out_specs=pl.BlockSpec((STEPS, BB), lambda p, b: (0, b)),
        out_shape=jax.ShapeDtypeStruct((STEPS, B), jnp.float32),
        scratch_shapes=[pltpu.VMEM((SROWS, 128), jnp.float32)],
    )(ahat, ahat, ahat, ahat)


def kernel(vi, delta_y, v_previous, x_input, w):
    x_t = jnp.transpose(x_input, (2, 1, 0))
    w16 = jnp.full((L,), w, jnp.float32)
    ahat = _sc_phase1(x_t, w16)
    out_t = _tc_phase2(ahat)
    return out_t.T
